# baseline (speedup 1.0000x reference)
import functools

import jax
import jax.numpy as jnp
from jax import lax
from jax.experimental import pallas as pl
from jax.experimental.pallas import tpu as pltpu
from jax.experimental.pallas import tpu_sc as plsc

VOCAB = 1000000
D = 16
F = 26
B = 16384

NC = 2
NS = 16
NW = NC * NS
L = 16

_mesh = plsc.VectorSubcoreMesh(core_axis_name="c", subcore_axis_name="s")


CB = 1152
NBLK_B = (VOCAB // 128) * 128 // CB
VOCAB_PAD = VOCAB // 128 * 128 + 128
TAIL_OFF = NBLK_B * CB
NBUF = 3
NG = (NBLK_B // NW + NBUF) // NBUF + 1


@functools.partial(
    pl.kernel,
    mesh=_mesh,
    out_type=jax.ShapeDtypeStruct((VOCAB_PAD * D,), jnp.float32),
    scratch_types=[
        pltpu.VMEM((D, CB), jnp.float32),
        pltpu.VMEM((D, CB), jnp.float32),
        pltpu.VMEM((D, CB), jnp.float32),
        pltpu.VMEM((CB * D,), jnp.float32),
        pltpu.VMEM((CB * D,), jnp.float32),
        pltpu.VMEM((CB * D,), jnp.float32),
        pltpu.SemaphoreType.DMA,
        pltpu.SemaphoreType.DMA,
        pltpu.SemaphoreType.DMA,
        pltpu.SemaphoreType.DMA,
        pltpu.SemaphoreType.DMA,
        pltpu.SemaphoreType.DMA,
    ],
    compiler_params=pltpu.CompilerParams(
        needs_layout_passes=False, use_tc_tiling_on_sc=True),
)
def _transpose_kernel(embt_hbm, tail_hbm, out_hbm, st0, st1, st2,
                      ob0, ob1, ob2, ss0, ss1, ss2, ws0, ws1, ws2):
    stage = (st0, st1, st2)
    obuf = (ob0, ob1, ob2)
    ssem = (ss0, ss1, ss2)
    wsem = (ws0, ws1, ws2)
    wid = lax.axis_index("s") * NC + lax.axis_index("c")
    lanes = lax.iota(jnp.int32, L)

    def transpose_rows(slot, nrows):
        idx_d = [lanes * D + d for d in range(D)]

        @plsc.parallel_loop(0, nrows // L, unroll=3)
        def cchunk(c):
            off = c * (L * D)
            for d in range(D):
                v = stage[slot][d, pl.ds(c * L, L)]
                plsc.store_scatter(obuf[slot], [idx_d[d] + off], v)

    @pl.when(wid == NW - 1)
    def _tail():
        pltpu.sync_copy(tail_hbm.at[pl.ds(0, 8), :],
                        st0.at[pl.ds(0, 8), pl.ds(0, 128)])
        pltpu.sync_copy(tail_hbm.at[pl.ds(8, 8), :],
                        st0.at[pl.ds(8, 8), pl.ds(0, 128)])
        transpose_rows(0, 128)
        pltpu.sync_copy(ob0.at[pl.ds(0, 128 * D)],
                        out_hbm.at[pl.ds(TAIL_OFF * D, 128 * D)])

    def issue_stage(bid, slot):
        col = bid * CB
        pltpu.make_async_copy(
            embt_hbm.at[pl.ds(0, 8), pl.ds(col, CB)],
            stage[slot].at[pl.ds(0, 8), :], ssem[slot]).start()
        pltpu.make_async_copy(
            embt_hbm.at[pl.ds(8, 8), pl.ds(col, CB)],
            stage[slot].at[pl.ds(8, 8), :], ssem[slot]).start()

    def wait_stage(slot):
        pltpu.make_async_copy(
            embt_hbm.at[pl.ds(0, 8), pl.ds(0, CB)],
            stage[slot].at[pl.ds(0, 8), :], ssem[slot]).wait()
        pltpu.make_async_copy(
            embt_hbm.at[pl.ds(8, 8), pl.ds(0, CB)],
            stage[slot].at[pl.ds(8, 8), :], ssem[slot]).wait()

    def wait_write(slot):
        pltpu.make_async_copy(
            obuf[slot], out_hbm.at[pl.ds(0, CB * D)],
            wsem[slot]).wait()

    for b in range(NBUF):
        bid0 = wid + NW * b

        @pl.when(bid0 < NBLK_B)
        def _prime(bid0=bid0, b=b):
            issue_stage(bid0, b)

    def grp(g, carry):
        for b in range(NBUF):
            j = g * NBUF + b
            bid = wid + NW * j
            ok = bid < NBLK_B

            @pl.when(ok)
            def _wait(b=b):
                wait_stage(b)

            @pl.when(jnp.logical_and(ok, j >= NBUF))
            def _wait_w(b=b):
                wait_write(b)

            @pl.when(ok)
            def _do(bid=bid, b=b):
                transpose_rows(b, CB)
                pltpu.make_async_copy(
                    obuf[b],
                    out_hbm.at[pl.ds(bid * (CB * D), CB * D)],
                    wsem[b]).start()

            @pl.when(bid + NW * NBUF < NBLK_B)
            def _next(bid=bid, b=b):
                issue_stage(bid + NW * NBUF, b)
        return carry

    lax.fori_loop(0, NG, grp, 0)

    for b in range(NBUF):
        @pl.when(wid + NW * b < NBLK_B)
        def _drain(b=b):
            wait_write(b)


ROWS_PER_W = B // NW
CH = 64
NCH = ROWS_PER_W // CH
IPC = CH * F
NIDX = IPC // 128
NCB = 4


@functools.partial(
    pl.kernel,
    mesh=_mesh,
    out_type=jax.ShapeDtypeStruct((B,), jnp.float32),
    scratch_types=[
        pltpu.VMEM((IPC,), jnp.int32),
        pltpu.VMEM((IPC,), jnp.int32),
        pltpu.VMEM((IPC,), jnp.int32),
        pltpu.VMEM((IPC,), jnp.int32),
        pltpu.VMEM((IPC, D), jnp.float32),
        pltpu.VMEM((IPC, D), jnp.float32),
        pltpu.VMEM((IPC, D), jnp.float32),
        pltpu.VMEM((IPC, D), jnp.float32),
        pltpu.VMEM((IPC + 2 * L,), jnp.float32),
        pltpu.VMEM((IPC + 2 * L,), jnp.float32),
        pltpu.VMEM((IPC + 2 * L,), jnp.float32),
        pltpu.VMEM((IPC + 2 * L,), jnp.float32),
        pltpu.VMEM((ROWS_PER_W,), jnp.float32),
        pltpu.VMEM((L,), jnp.float32),
        pltpu.SemaphoreType.DMA,
        pltpu.SemaphoreType.DMA,
        pltpu.SemaphoreType.DMA,
        pltpu.SemaphoreType.DMA,
    ],
    compiler_params=pltpu.CompilerParams(
        needs_layout_passes=False, use_tc_tiling_on_sc=False),
)
def _fm_kernel(x_hbm, emb_hbm, lin_hbm, bias_hbm, out_hbm,
               idx0, idx1, idx2, idx3, rows0, rows1, rows2, rows3,
               lin0, lin1, lin2, lin3, out_v, bias_v,
               gs0, gs1, gs2, gs3):
    idx = (idx0, idx1, idx2, idx3)
    rows = (rows0, rows1, rows2, rows3)
    lin = (lin0, lin1, lin2, lin3)
    gsem = (gs0, gs1, gs2, gs3)
    wid = lax.axis_index("s") * NC + lax.axis_index("c")
    ibase = wid * (ROWS_PER_W * F)

    pltpu.sync_copy(bias_hbm, bias_v)
    bias_vec = bias_v[...]
    lanes = lax.iota(jnp.int32, L)

    def issue_gathers(c, slot):
        off = ibase + c * IPC
        pltpu.sync_copy(x_hbm.at[pl.ds(off, IPC)], idx[slot])
        pltpu.make_async_copy(
            emb_hbm.at[idx[slot]], rows[slot], gsem[slot]).start()
        pltpu.make_async_copy(
            lin_hbm.at[idx[slot]], lin[slot].at[pl.ds(0, IPC)],
            gsem[slot]).start()

    def wait_gathers(slot):
        pltpu.make_async_copy(
            emb_hbm.at[idx[slot]], rows[slot], gsem[slot]).wait()
        pltpu.make_async_copy(
            lin_hbm.at[idx[slot]], lin[slot].at[pl.ds(0, IPC)],
            gsem[slot]).wait()

    def compute_chunk(c, slot):
        rows_v = rows[slot]
        lin_v = lin[slot]

        def group_body(g, carry2):
            rowbase = g * L

            def row_body(j, cross_vec):
                p = (rowbase + j) * F
                v0 = rows_v[p, :]
                v1 = rows_v[p + 1, :]
                s0, q0 = v0, v0 * v0
                s1, q1 = v1, v1 * v1
                for f in range(2, F, 2):
                    va = rows_v[p + f, :]
                    s0 = s0 + va
                    q0 = q0 + va * va
                for f in range(3, F, 2):
                    vb = rows_v[p + f, :]
                    s1 = s1 + vb
                    q1 = q1 + vb * vb
                s = s0 + s1
                q = q0 + q1
                la = lin_v[pl.ds(p, L)]
                lb = lin_v[pl.ds(p + L, L)]
                lb = jnp.where(lanes < F - L, lb, 0.0)
                total_vec = la + lb + 0.5 * (s * s - q)
                red = jnp.sum(total_vec)
                return jnp.where(lanes == j, red, cross_vec)

            cross_vec = lax.fori_loop(0, L, row_body,
                                      jnp.zeros((L,), jnp.float32))
            z = cross_vec + bias_vec
            yv = 1.0 / (1.0 + jnp.exp(-z))
            out_v[pl.ds(c * CH + rowbase, L)] = yv
            return carry2

        lax.fori_loop(0, CH // L, group_body, 0)

    for pc in range(NCB - 1):
        issue_gathers(pc, pc)

    def grp(g, carry):
        for b in range(NCB):
            c = g * NCB + b
            wait_gathers(b)

            @pl.when(c + NCB - 1 < NCH)
            def _nxt(c=c, b=b):
                issue_gathers(c + NCB - 1, (b + NCB - 1) % NCB)

            compute_chunk(c, b)
        return carry

    lax.fori_loop(0, NCH // NCB, grp, 0)
    pltpu.sync_copy(out_v, out_hbm.at[pl.ds(wid * ROWS_PER_W, ROWS_PER_W)])


def kernel(X, y, emb_table, lin_w, bias):
    emb_t = emb_table.T
    tail = jnp.pad(emb_table[TAIL_OFF:], ((0, 64), (0, 0))).T
    flat = _transpose_kernel(emb_t, tail)
    emb_lin = flat.reshape(VOCAB_PAD, D)
    x_flat = X.reshape(-1).astype(jnp.int32)
    lin_flat = lin_w.reshape(-1)
    bias16 = jnp.broadcast_to(bias.astype(jnp.float32), (L,))
    y_pred = _fm_kernel(x_flat, emb_lin, lin_flat, bias16)
    return (y.reshape(B, 1), y_pred.reshape(B, 1))


# speedup vs baseline: 1.2151x; 1.2151x over previous
import functools

import jax
import jax.numpy as jnp
from jax import lax
from jax.experimental import pallas as pl
from jax.experimental.pallas import tpu as pltpu
from jax.experimental.pallas import tpu_sc as plsc

VOCAB = 1000000
D = 16
F = 26
B = 16384

NC = 2
NS = 16
NW = NC * NS
L = 16

_mesh = plsc.VectorSubcoreMesh(core_axis_name="c", subcore_axis_name="s")


CB = 1152
NBLK_B = (VOCAB // 128) * 128 // CB
VOCAB_PAD = VOCAB // 128 * 128 + 128
TAIL_OFF = NBLK_B * CB
NBUF = 3
NG = (NBLK_B // NW + NBUF) // NBUF + 1


@functools.partial(
    pl.kernel,
    mesh=_mesh,
    out_type=jax.ShapeDtypeStruct((VOCAB_PAD * D,), jnp.float32),
    scratch_types=[
        pltpu.VMEM((D, CB), jnp.float32),
        pltpu.VMEM((D, CB), jnp.float32),
        pltpu.VMEM((D, CB), jnp.float32),
        pltpu.VMEM((CB * D,), jnp.float32),
        pltpu.VMEM((CB * D,), jnp.float32),
        pltpu.VMEM((CB * D,), jnp.float32),
        pltpu.SemaphoreType.DMA,
        pltpu.SemaphoreType.DMA,
        pltpu.SemaphoreType.DMA,
        pltpu.SemaphoreType.DMA,
        pltpu.SemaphoreType.DMA,
        pltpu.SemaphoreType.DMA,
    ],
    compiler_params=pltpu.CompilerParams(
        needs_layout_passes=False, use_tc_tiling_on_sc=True),
)
def _transpose_kernel(embt_hbm, tail_hbm, out_hbm, st0, st1, st2,
                      ob0, ob1, ob2, ss0, ss1, ss2, ws0, ws1, ws2):
    stage = (st0, st1, st2)
    obuf = (ob0, ob1, ob2)
    ssem = (ss0, ss1, ss2)
    wsem = (ws0, ws1, ws2)
    wid = lax.axis_index("s") * NC + lax.axis_index("c")
    lanes = lax.iota(jnp.int32, L)

    def transpose_rows(slot, nrows):
        idx_d = [lanes * D + d for d in range(D)]

        @plsc.parallel_loop(0, nrows // L, unroll=2)
        def cchunk(c):
            off = c * (L * D)
            for d in range(D):
                v = stage[slot][d, pl.ds(c * L, L)]
                plsc.store_scatter(obuf[slot], [idx_d[d] + off], v)

    @pl.when(wid == NW - 1)
    def _tail():
        pltpu.sync_copy(tail_hbm.at[pl.ds(0, 8), :],
                        st0.at[pl.ds(0, 8), pl.ds(0, 128)])
        pltpu.sync_copy(tail_hbm.at[pl.ds(8, 8), :],
                        st0.at[pl.ds(8, 8), pl.ds(0, 128)])
        transpose_rows(0, 128)
        pltpu.sync_copy(ob0.at[pl.ds(0, 128 * D)],
                        out_hbm.at[pl.ds(TAIL_OFF * D, 128 * D)])

    def issue_stage(bid, slot):
        col = bid * CB
        pltpu.make_async_copy(
            embt_hbm.at[pl.ds(0, 8), pl.ds(col, CB)],
            stage[slot].at[pl.ds(0, 8), :], ssem[slot]).start()
        pltpu.make_async_copy(
            embt_hbm.at[pl.ds(8, 8), pl.ds(col, CB)],
            stage[slot].at[pl.ds(8, 8), :], ssem[slot]).start()

    def wait_stage(slot):
        pltpu.make_async_copy(
            embt_hbm.at[pl.ds(0, 8), pl.ds(0, CB)],
            stage[slot].at[pl.ds(0, 8), :], ssem[slot]).wait()
        pltpu.make_async_copy(
            embt_hbm.at[pl.ds(8, 8), pl.ds(0, CB)],
            stage[slot].at[pl.ds(8, 8), :], ssem[slot]).wait()

    def wait_write(slot):
        pltpu.make_async_copy(
            obuf[slot], out_hbm.at[pl.ds(0, CB * D)],
            wsem[slot]).wait()

    for b in range(NBUF):
        bid0 = wid + NW * b

        @pl.when(bid0 < NBLK_B)
        def _prime(bid0=bid0, b=b):
            issue_stage(bid0, b)

    def grp(g, carry):
        for b in range(NBUF):
            j = g * NBUF + b
            bid = wid + NW * j
            ok = bid < NBLK_B

            @pl.when(ok)
            def _wait(b=b):
                wait_stage(b)

            @pl.when(jnp.logical_and(ok, j >= NBUF))
            def _wait_w(b=b):
                wait_write(b)

            @pl.when(ok)
            def _do(bid=bid, b=b):
                transpose_rows(b, CB)
                pltpu.make_async_copy(
                    obuf[b],
                    out_hbm.at[pl.ds(bid * (CB * D), CB * D)],
                    wsem[b]).start()

            @pl.when(bid + NW * NBUF < NBLK_B)
            def _next(bid=bid, b=b):
                issue_stage(bid + NW * NBUF, b)
        return carry

    lax.fori_loop(0, NG, grp, 0)

    for b in range(NBUF):
        @pl.when(wid + NW * b < NBLK_B)
        def _drain(b=b):
            wait_write(b)


ROWS_PER_W = B // NW
CH = 64
NCH = ROWS_PER_W // CH
IPC = CH * F
NIDX = IPC // 128
NCB = 4


@functools.partial(
    pl.kernel,
    mesh=_mesh,
    out_type=jax.ShapeDtypeStruct((B,), jnp.float32),
    scratch_types=[
        pltpu.VMEM((IPC,), jnp.int32),
        pltpu.VMEM((IPC,), jnp.int32),
        pltpu.VMEM((IPC,), jnp.int32),
        pltpu.VMEM((IPC,), jnp.int32),
        pltpu.VMEM((IPC, D), jnp.float32),
        pltpu.VMEM((IPC, D), jnp.float32),
        pltpu.VMEM((IPC, D), jnp.float32),
        pltpu.VMEM((IPC, D), jnp.float32),
        pltpu.VMEM((IPC + 2 * L,), jnp.float32),
        pltpu.VMEM((IPC + 2 * L,), jnp.float32),
        pltpu.VMEM((IPC + 2 * L,), jnp.float32),
        pltpu.VMEM((IPC + 2 * L,), jnp.float32),
        pltpu.VMEM((ROWS_PER_W,), jnp.float32),
        pltpu.VMEM((L,), jnp.float32),
        pltpu.SemaphoreType.DMA,
        pltpu.SemaphoreType.DMA,
        pltpu.SemaphoreType.DMA,
        pltpu.SemaphoreType.DMA,
    ],
    compiler_params=pltpu.CompilerParams(
        needs_layout_passes=False, use_tc_tiling_on_sc=False),
)
def _fm_kernel(x_hbm, emb_hbm, lin_hbm, bias_hbm, out_hbm,
               idx0, idx1, idx2, idx3, rows0, rows1, rows2, rows3,
               lin0, lin1, lin2, lin3, out_v, bias_v,
               gs0, gs1, gs2, gs3):
    idx = (idx0, idx1, idx2, idx3)
    rows = (rows0, rows1, rows2, rows3)
    lin = (lin0, lin1, lin2, lin3)
    gsem = (gs0, gs1, gs2, gs3)
    wid = lax.axis_index("s") * NC + lax.axis_index("c")
    ibase = wid * (ROWS_PER_W * F)

    pltpu.sync_copy(bias_hbm, bias_v)
    bias_vec = bias_v[...]
    lanes = lax.iota(jnp.int32, L)

    def issue_gathers(c, slot):
        off = ibase + c * IPC
        pltpu.sync_copy(x_hbm.at[pl.ds(off, IPC)], idx[slot])
        pltpu.make_async_copy(
            emb_hbm.at[idx[slot]], rows[slot], gsem[slot]).start()
        pltpu.make_async_copy(
            lin_hbm.at[idx[slot]], lin[slot].at[pl.ds(0, IPC)],
            gsem[slot]).start()

    def wait_gathers(slot):
        pltpu.make_async_copy(
            emb_hbm.at[idx[slot]], rows[slot], gsem[slot]).wait()
        pltpu.make_async_copy(
            lin_hbm.at[idx[slot]], lin[slot].at[pl.ds(0, IPC)],
            gsem[slot]).wait()

    def compute_chunk(c, slot):
        rows_v = rows[slot]
        lin_v = lin[slot]

        def group_body(g, carry2):
            rowbase = g * L

            def row_body(j, cross_vec):
                p = (rowbase + j) * F
                v0 = rows_v[p, :]
                v1 = rows_v[p + 1, :]
                s0, q0 = v0, v0 * v0
                s1, q1 = v1, v1 * v1
                for f in range(2, F, 2):
                    va = rows_v[p + f, :]
                    s0 = s0 + va
                    q0 = q0 + va * va
                for f in range(3, F, 2):
                    vb = rows_v[p + f, :]
                    s1 = s1 + vb
                    q1 = q1 + vb * vb
                s = s0 + s1
                q = q0 + q1
                la = lin_v[pl.ds(p, L)]
                lb = lin_v[pl.ds(p + L, L)]
                lb = jnp.where(lanes < F - L, lb, 0.0)
                total_vec = la + lb + 0.5 * (s * s - q)
                red = jnp.sum(total_vec)
                return jnp.where(lanes == j, red, cross_vec)

            cross_vec = lax.fori_loop(0, L, row_body,
                                      jnp.zeros((L,), jnp.float32))
            z = cross_vec + bias_vec
            yv = 1.0 / (1.0 + jnp.exp(-z))
            out_v[pl.ds(c * CH + rowbase, L)] = yv
            return carry2

        lax.fori_loop(0, CH // L, group_body, 0)

    for pc in range(NCB - 1):
        issue_gathers(pc, pc)

    def grp(g, carry):
        for b in range(NCB):
            c = g * NCB + b
            wait_gathers(b)

            @pl.when(c + NCB - 1 < NCH)
            def _nxt(c=c, b=b):
                issue_gathers(c + NCB - 1, (b + NCB - 1) % NCB)

            compute_chunk(c, b)
        return carry

    lax.fori_loop(0, NCH // NCB, grp, 0)
    pltpu.sync_copy(out_v, out_hbm.at[pl.ds(wid * ROWS_PER_W, ROWS_PER_W)])


def kernel(X, y, emb_table, lin_w, bias):
    emb_t = emb_table.T
    tail = jnp.pad(emb_table[TAIL_OFF:], ((0, 64), (0, 0))).T
    flat = _transpose_kernel(emb_t, tail)
    emb_lin = flat.reshape(VOCAB_PAD, D)
    x_flat = X.reshape(-1).astype(jnp.int32)
    lin_flat = lin_w.reshape(-1)
    bias16 = jnp.broadcast_to(bias.astype(jnp.float32), (L,))
    y_pred = _fm_kernel(x_flat, emb_lin, lin_flat, bias16)
    return (y.reshape(B, 1), y_pred.reshape(B, 1))
